# baseline (speedup 1.0000x reference)
import functools

import jax
import jax.numpy as jnp
from jax import lax
from jax.experimental import pallas as pl
from jax.experimental.pallas import tpu as pltpu
from jax.experimental.pallas import tpu_sc as plsc

B = 32
N = 512
E = 8192
EPB = 2 * E
ECHUNK = 8192
QROWS = 128


def _sc_body(src_hbm, dst_hbm, idx_hbm, table_hbm,
             a_hbm, emb_hbm,
             src_v, dst_v, acc_v, idx_v, rows_v, sem, sem2, sem3):
    f32 = jnp.float32
    wid = lax.axis_index("s") * 2 + lax.axis_index("c")
    j = wid // 2
    q0 = 2 * lax.rem(wid, 2)

    ce1 = pltpu.async_copy(src_hbm.at[j], src_v, sem)
    ce2 = pltpu.async_copy(dst_hbm.at[j], dst_v, sem2)

    @pl.when(wid < 8)
    def _():
        base = wid * 8
        pltpu.sync_copy(idx_hbm.at[pl.ds(base, 8)], idx_v)
        pltpu.async_copy(table_hbm.at[idx_v], rows_v, sem3).wait()
        pltpu.sync_copy(rows_v, emb_hbm.at[pl.ds(base, 8)])

    ce1.wait()
    ce2.wait()

    zeros16 = jnp.zeros((16,), f32)
    ones16 = jnp.ones((16,), f32)

    for t in range(2):
        qlo = (q0 + t) * QROWS

        def zstep(i, carry):
            for u in range(16):
                acc_v[pl.ds(i * 256 + u * 16, 16)] = zeros16
            return carry

        lax.fori_loop(0, QROWS * N // 256, zstep, 0)

        def step(i, carry):
            for u in range(8):
                off = i * 128 + u * 16
                s16 = src_v[pl.ds(off, 16)]
                d16 = dst_v[pl.ds(off, 16)]
                dl = d16 - qlo
                inr = dl.astype(jnp.uint32) < jnp.uint32(QROWS)
                flat = jnp.where(inr, dl * N + s16, 0)
                vals = jnp.where(inr, ones16, 0.0)
                plsc.addupdate_scatter(acc_v, [flat], vals)
            return carry

        lax.fori_loop(0, EPB // 128, step, 0)
        pltpu.sync_copy(acc_v, a_hbm.at[wid * 2 + t])


def _sc_sparse(src, dst, idx, table):
    mesh = plsc.VectorSubcoreMesh(core_axis_name="c", subcore_axis_name="s")
    f32 = jnp.float32
    return pl.kernel(
        _sc_body,
        out_type=(
            jax.ShapeDtypeStruct((64, QROWS * N), f32),
            jax.ShapeDtypeStruct((64, N), f32),
        ),
        mesh=mesh,
        scratch_types=(
            pltpu.VMEM((EPB,), jnp.int32),
            pltpu.VMEM((EPB,), jnp.int32),
            pltpu.VMEM((QROWS * N,), f32),
            pltpu.VMEM((8,), jnp.int32),
            pltpu.VMEM((8, N), f32),
            pltpu.SemaphoreType.DMA,
            pltpu.SemaphoreType.DMA,
            pltpu.SemaphoreType.DMA,
        ),
        compiler_params=pltpu.CompilerParams(needs_layout_passes=False),
    )(src, dst, idx, table)


def _tc_body(ea1, ed1, ea2, ed2, nf1, nf2, a_r,
             w1, b1, w2, b2, m1, c1, m2, c2, woutT, outb, out_ref):
    f32 = jnp.float32
    A = a_r[0]

    deg = 1.0 + jnp.sum(A, axis=1)
    dinv = lax.rsqrt(deg)
    dinv2 = 1.0 / deg

    w10 = w1[0, 0, :]
    w11 = w1[0, 1, :]
    w12 = w1[0, 2, :]
    a1 = ea1[0, 0, :]
    d1 = ed1[0, 0, :]
    n1 = nf1[0, 0, :]
    a2 = ea2[0, 0, :]
    d2 = ed2[0, 0, :]
    n2 = nf2[0, 0, :]
    xw1_j = (a1[:, None] * w10[None, :] + d1[:, None] * w11[None, :]
             + n1[:, None] * w12[None, :])
    xw1_p = (a2[:, None] * w10[None, :] + d2[:, None] * w11[None, :]
             + n2[:, None] * w12[None, :])

    bf16 = jnp.bfloat16
    b1v = b1[0, 0, :]
    h1_j = jnp.maximum(xw1_j + b1v[None, :], 0.0)
    W2bf = w2[0].astype(bf16)
    h1w2_j = jnp.dot(h1_j.astype(bf16), W2bf,
                     preferred_element_type=f32)

    z = jnp.concatenate([xw1_j, h1w2_j], axis=1)
    agg = jnp.dot(A.astype(bf16), z.astype(bf16), preferred_element_type=f32)

    h1_p = jnp.maximum(dinv2[:, None] * xw1_p + dinv[:, None] * agg[:, :256]
                       + b1v[None, :], 0.0)
    b2v = b2[0, 0, :]
    h2_j = h1w2_j + b2v[None, :]
    h2_p = (dinv2[:, None] * jnp.dot(h1_p.astype(bf16), W2bf,
                                     preferred_element_type=f32)
            + dinv[:, None] * agg[:, 256:] + b2v[None, :])

    c1v = c1[0, 0, :]
    c2v = c2[0, 0, :]
    wo = woutT[0]
    ob = outb[0, 0, :]
    m1bf = m1[0].astype(bf16)
    m2bf = m2[0].astype(bf16)

    def head(h2):
        t1 = jnp.maximum(jnp.dot(h2.astype(bf16), m1bf,
                                 preferred_element_type=f32)
                         + c1v[None, :], 0.0)
        t2 = jnp.maximum(jnp.dot(t1.astype(bf16), m2bf,
                                 preferred_element_type=f32)
                         + c2v[None, :], 0.0)
        s = jnp.sum(t2 * wo, axis=1) + ob
        return 1.0 / (1.0 + jnp.exp(-s))

    out_ref[0, 0, :] = head(h2_j)
    out_ref[0, 1, :] = head(h2_p)


def kernel(actions, node_features, edge_index, emb_table,
           conv1_W, conv1_b, conv2_W, conv2_b,
           mlp1_W, mlp1_b, mlp2_W, mlp2_b, out_W, out_b):
    f32 = jnp.float32
    idx = jnp.concatenate([actions[:, 0], actions[:, 1]]).astype(jnp.int32)
    src = edge_index[:16].reshape(16, EPB)
    dst = edge_index[16:].reshape(16, EPB)

    a_q, emb_rows = _sc_sparse(src, dst, idx, emb_table)
    a_mat = a_q.reshape(16, N, N)
    emb3 = emb_rows.reshape(64, 1, N)
    nf3 = node_features

    w1 = conv1_W.reshape(1, 3, 256)
    b1 = conv1_b.reshape(1, 1, 256)
    w2 = conv2_W.reshape(1, 256, 128)
    b2 = conv2_b.reshape(1, 1, 128)
    m1 = mlp1_W.reshape(1, 128, 256)
    c1 = mlp1_b.reshape(1, 1, 256)
    m2 = mlp2_W.reshape(1, 256, 256)
    c2 = mlp2_b.reshape(1, 1, 256)
    woutT = out_W.reshape(1, 1, 256)
    outb = jnp.broadcast_to(out_b.reshape(1, 1, 1), (1, 1, N))

    const = lambda j: (0, 0, 0)
    out = pl.pallas_call(
        _tc_body,
        grid=(16,),
        in_specs=[
            pl.BlockSpec((1, 1, N), lambda j: (j, 0, 0)),
            pl.BlockSpec((1, 1, N), lambda j: (j + 32, 0, 0)),
            pl.BlockSpec((1, 1, N), lambda j: (j + 16, 0, 0)),
            pl.BlockSpec((1, 1, N), lambda j: (j + 48, 0, 0)),
            pl.BlockSpec((1, 1, N), lambda j: (j, 0, 0)),
            pl.BlockSpec((1, 1, N), lambda j: (j + 16, 0, 0)),
            pl.BlockSpec((1, N, N), lambda j: (j, 0, 0)),
            pl.BlockSpec((1, 3, 256), const),
            pl.BlockSpec((1, 1, 256), const),
            pl.BlockSpec((1, 256, 128), const),
            pl.BlockSpec((1, 1, 128), const),
            pl.BlockSpec((1, 128, 256), const),
            pl.BlockSpec((1, 1, 256), const),
            pl.BlockSpec((1, 256, 256), const),
            pl.BlockSpec((1, 1, 256), const),
            pl.BlockSpec((1, 1, 256), const),
            pl.BlockSpec((1, 1, N), const),
        ],
        out_specs=pl.BlockSpec((1, 2, N), lambda j: (j, 0, 0)),
        out_shape=jax.ShapeDtypeStruct((16, 2, N), f32),
        compiler_params=pltpu.CompilerParams(
            dimension_semantics=("arbitrary",),
        ),
    )(emb3, emb3, emb3, emb3, nf3, nf3, a_mat,
      w1, b1, w2, b2, m1, c1, m2, c2, woutT, outb)
    return out.transpose(1, 0, 2).reshape(B, N)


# speedup vs baseline: 1.3196x; 1.3196x over previous
import functools

import jax
import jax.numpy as jnp
from jax import lax
from jax.experimental import pallas as pl
from jax.experimental.pallas import tpu as pltpu
from jax.experimental.pallas import tpu_sc as plsc

B = 32
N = 512
E = 8192
EPB = 2 * E
ECHUNK = 8192
QROWS = 128


def _sc_body(src_hbm, dst_hbm, idx_hbm, table_hbm,
             a_hbm, emb_hbm,
             src_v, dst_v, acc_v, idx_v, rows_v, sem, sem2, sem3):
    f32 = jnp.float32
    wid = lax.axis_index("s") * 2 + lax.axis_index("c")
    j = wid // 2
    q0 = 2 * lax.rem(wid, 2)

    ce1 = pltpu.async_copy(src_hbm.at[j], src_v, sem)
    ce2 = pltpu.async_copy(dst_hbm.at[j], dst_v, sem2)

    @pl.when(wid < 8)
    def _():
        base = wid * 8
        pltpu.sync_copy(idx_hbm.at[pl.ds(base, 8)], idx_v)
        pltpu.async_copy(table_hbm.at[idx_v], rows_v, sem3).wait()
        pltpu.sync_copy(rows_v, emb_hbm.at[pl.ds(base, 8)])

    ce1.wait()
    ce2.wait()

    zeros16 = jnp.zeros((16,), f32)
    ones16 = jnp.ones((16,), f32)

    for t in range(2):
        qlo = (q0 + t) * QROWS

        def zrow(i, carry):
            for u in range(N // 16):
                acc_v[i, pl.ds(u * 16, 16)] = zeros16
            return carry

        lax.fori_loop(0, QROWS, zrow, 0)

        def step(i, carry):
            for u in range(8):
                off = i * 128 + u * 16
                s16 = src_v[pl.ds(off, 16)]
                d16 = dst_v[pl.ds(off, 16)]
                dl = d16 - qlo
                inr = dl.astype(jnp.uint32) < jnp.uint32(QROWS)
                r = jnp.where(inr, dl, 0)
                vals = jnp.where(inr, ones16, 0.0)
                plsc.addupdate_scatter(acc_v, [r, s16], vals)
            return carry

        lax.fori_loop(0, EPB // 128, step, 0)
        pltpu.sync_copy(acc_v, a_hbm.at[j, pl.ds(qlo, QROWS)])


def _sc_sparse(src, dst, idx, table):
    mesh = plsc.VectorSubcoreMesh(core_axis_name="c", subcore_axis_name="s")
    f32 = jnp.float32
    return pl.kernel(
        _sc_body,
        out_type=(
            jax.ShapeDtypeStruct((16, N, N), f32),
            jax.ShapeDtypeStruct((64, N), f32),
        ),
        mesh=mesh,
        scratch_types=(
            pltpu.VMEM((EPB,), jnp.int32),
            pltpu.VMEM((EPB,), jnp.int32),
            pltpu.VMEM((QROWS, N), f32),
            pltpu.VMEM((8,), jnp.int32),
            pltpu.VMEM((8, N), f32),
            pltpu.SemaphoreType.DMA,
            pltpu.SemaphoreType.DMA,
            pltpu.SemaphoreType.DMA,
        ),
        compiler_params=pltpu.CompilerParams(needs_layout_passes=False),
    )(src, dst, idx, table)


def _tc_body(ea1, ed1, ea2, ed2, nf1, nf2, a_r,
             w1, b1, w2, b2, m1, c1, m2, c2, woutT, outb, out_ref):
    f32 = jnp.float32
    A = a_r[0]

    deg = 1.0 + jnp.sum(A, axis=1)
    dinv = lax.rsqrt(deg)
    dinv2 = 1.0 / deg

    w10 = w1[0, 0, :]
    w11 = w1[0, 1, :]
    w12 = w1[0, 2, :]
    a1 = ea1[0, 0, :]
    d1 = ed1[0, 0, :]
    n1 = nf1[0, 0, :]
    a2 = ea2[0, 0, :]
    d2 = ed2[0, 0, :]
    n2 = nf2[0, 0, :]
    xw1_j = (a1[:, None] * w10[None, :] + d1[:, None] * w11[None, :]
             + n1[:, None] * w12[None, :])
    xw1_p = (a2[:, None] * w10[None, :] + d2[:, None] * w11[None, :]
             + n2[:, None] * w12[None, :])

    bf16 = jnp.bfloat16
    b1v = b1[0, 0, :]
    h1_j = jnp.maximum(xw1_j + b1v[None, :], 0.0)
    W2bf = w2[0].astype(bf16)
    h1w2_j = jnp.dot(h1_j.astype(bf16), W2bf,
                     preferred_element_type=f32)

    z = jnp.concatenate([xw1_j, h1w2_j], axis=1)
    agg = jnp.dot(A.astype(bf16), z.astype(bf16), preferred_element_type=f32)

    h1_p = jnp.maximum(dinv2[:, None] * xw1_p + dinv[:, None] * agg[:, :256]
                       + b1v[None, :], 0.0)
    b2v = b2[0, 0, :]
    h2_j = h1w2_j + b2v[None, :]
    h2_p = (dinv2[:, None] * jnp.dot(h1_p.astype(bf16), W2bf,
                                     preferred_element_type=f32)
            + dinv[:, None] * agg[:, 256:] + b2v[None, :])

    c1v = c1[0, 0, :]
    c2v = c2[0, 0, :]
    wo = woutT[0]
    ob = outb[0, 0, :]
    m1bf = m1[0].astype(bf16)
    m2bf = m2[0].astype(bf16)

    def head(h2):
        t1 = jnp.maximum(jnp.dot(h2.astype(bf16), m1bf,
                                 preferred_element_type=f32)
                         + c1v[None, :], 0.0)
        t2 = jnp.maximum(jnp.dot(t1.astype(bf16), m2bf,
                                 preferred_element_type=f32)
                         + c2v[None, :], 0.0)
        s = jnp.sum(t2 * wo, axis=1) + ob
        return 1.0 / (1.0 + jnp.exp(-s))

    out_ref[0, 0, :] = head(h2_j)
    out_ref[0, 1, :] = head(h2_p)


def kernel(actions, node_features, edge_index, emb_table,
           conv1_W, conv1_b, conv2_W, conv2_b,
           mlp1_W, mlp1_b, mlp2_W, mlp2_b, out_W, out_b):
    f32 = jnp.float32
    idx = jnp.concatenate([actions[:, 0], actions[:, 1]]).astype(jnp.int32)
    src = edge_index[:16].reshape(16, EPB)
    dst = edge_index[16:].reshape(16, EPB)

    a_mat, emb_rows = _sc_sparse(src, dst, idx, emb_table)
    emb3 = emb_rows.reshape(64, 1, N)
    nf3 = node_features

    w1 = conv1_W.reshape(1, 3, 256)
    b1 = conv1_b.reshape(1, 1, 256)
    w2 = conv2_W.reshape(1, 256, 128)
    b2 = conv2_b.reshape(1, 1, 128)
    m1 = mlp1_W.reshape(1, 128, 256)
    c1 = mlp1_b.reshape(1, 1, 256)
    m2 = mlp2_W.reshape(1, 256, 256)
    c2 = mlp2_b.reshape(1, 1, 256)
    woutT = out_W.reshape(1, 1, 256)
    outb = jnp.broadcast_to(out_b.reshape(1, 1, 1), (1, 1, N))

    const = lambda j: (0, 0, 0)
    out = pl.pallas_call(
        _tc_body,
        grid=(16,),
        in_specs=[
            pl.BlockSpec((1, 1, N), lambda j: (j, 0, 0)),
            pl.BlockSpec((1, 1, N), lambda j: (j + 32, 0, 0)),
            pl.BlockSpec((1, 1, N), lambda j: (j + 16, 0, 0)),
            pl.BlockSpec((1, 1, N), lambda j: (j + 48, 0, 0)),
            pl.BlockSpec((1, 1, N), lambda j: (j, 0, 0)),
            pl.BlockSpec((1, 1, N), lambda j: (j + 16, 0, 0)),
            pl.BlockSpec((1, N, N), lambda j: (j, 0, 0)),
            pl.BlockSpec((1, 3, 256), const),
            pl.BlockSpec((1, 1, 256), const),
            pl.BlockSpec((1, 256, 128), const),
            pl.BlockSpec((1, 1, 128), const),
            pl.BlockSpec((1, 128, 256), const),
            pl.BlockSpec((1, 1, 256), const),
            pl.BlockSpec((1, 256, 256), const),
            pl.BlockSpec((1, 1, 256), const),
            pl.BlockSpec((1, 1, 256), const),
            pl.BlockSpec((1, 1, N), const),
        ],
        out_specs=pl.BlockSpec((1, 2, N), lambda j: (j, 0, 0)),
        out_shape=jax.ShapeDtypeStruct((16, 2, N), f32),
        compiler_params=pltpu.CompilerParams(
            dimension_semantics=("arbitrary",),
        ),
    )(emb3, emb3, emb3, emb3, nf3, nf3, a_mat,
      w1, b1, w2, b2, m1, c1, m2, c2, woutT, outb)
    return out.transpose(1, 0, 2).reshape(B, N)


# speedup vs baseline: 1.3468x; 1.0207x over previous
import functools

import jax
import jax.numpy as jnp
from jax import lax
from jax.experimental import pallas as pl
from jax.experimental.pallas import tpu as pltpu
from jax.experimental.pallas import tpu_sc as plsc

B = 32
N = 512
E = 8192
EPB = 2 * E
ECHUNK = 8192
QROWS = 128


def _sc_body(src_hbm, dst_hbm, idx_hbm, table_hbm,
             a_hbm, emb_hbm,
             src_v, dst_v, acc_v, idx_v, rows_v, sem, sem2, sem3):
    f32 = jnp.float32
    wid = lax.axis_index("s") * 2 + lax.axis_index("c")
    j = wid // 2
    q0 = 2 * lax.rem(wid, 2)

    ce1 = pltpu.async_copy(src_hbm.at[j], src_v, sem)
    ce2 = pltpu.async_copy(dst_hbm.at[j], dst_v, sem2)

    @pl.when(wid < 8)
    def _():
        base = wid * 8
        pltpu.sync_copy(idx_hbm.at[pl.ds(base, 8)], idx_v)
        pltpu.async_copy(table_hbm.at[idx_v], rows_v, sem3)

    zeros16 = jnp.zeros((16,), f32)
    ones16 = jnp.ones((16,), f32)

    def zrow(i, carry):
        for u in range(N // 16):
            acc_v[i, pl.ds(u * 16, 16)] = zeros16
        return carry

    lax.fori_loop(0, QROWS, zrow, 0)
    ce1.wait()
    ce2.wait()

    for t in range(2):
        qlo = (q0 + t) * QROWS
        if t:
            lax.fori_loop(0, QROWS, zrow, 0)

        def step(i, carry):
            for u in range(8):
                off = i * 128 + u * 16
                s16 = src_v[pl.ds(off, 16)]
                d16 = dst_v[pl.ds(off, 16)]
                dl = d16 - qlo
                inr = dl.astype(jnp.uint32) < jnp.uint32(QROWS)
                r = jnp.where(inr, dl, 0)
                vals = jnp.where(inr, ones16, 0.0)
                plsc.addupdate_scatter(acc_v, [r, s16], vals)
            return carry

        lax.fori_loop(0, EPB // 128, step, 0)
        pltpu.sync_copy(acc_v, a_hbm.at[j, pl.ds(qlo, QROWS)])

    @pl.when(wid < 8)
    def _():
        pltpu.make_async_copy(table_hbm.at[idx_v], rows_v, sem3).wait()
        pltpu.sync_copy(rows_v, emb_hbm.at[pl.ds(wid * 8, 8)])


def _sc_sparse(src, dst, idx, table):
    mesh = plsc.VectorSubcoreMesh(core_axis_name="c", subcore_axis_name="s")
    f32 = jnp.float32
    return pl.kernel(
        _sc_body,
        out_type=(
            jax.ShapeDtypeStruct((16, N, N), f32),
            jax.ShapeDtypeStruct((64, N), f32),
        ),
        mesh=mesh,
        scratch_types=(
            pltpu.VMEM((EPB,), jnp.int32),
            pltpu.VMEM((EPB,), jnp.int32),
            pltpu.VMEM((QROWS, N), f32),
            pltpu.VMEM((8,), jnp.int32),
            pltpu.VMEM((8, N), f32),
            pltpu.SemaphoreType.DMA,
            pltpu.SemaphoreType.DMA,
            pltpu.SemaphoreType.DMA,
        ),
        compiler_params=pltpu.CompilerParams(needs_layout_passes=False),
    )(src, dst, idx, table)


def _tc_body(ea1, ed1, ea2, ed2, nf1, nf2, a_r,
             w1, b1, w2, b2, m1, c1, m2, c2, woutT, outb, out_ref):
    f32 = jnp.float32
    A = a_r[0]

    deg = 1.0 + jnp.sum(A, axis=1)
    dinv = lax.rsqrt(deg)
    dinv2 = 1.0 / deg

    w10 = w1[0, 0, :]
    w11 = w1[0, 1, :]
    w12 = w1[0, 2, :]
    a1 = ea1[0, 0, :]
    d1 = ed1[0, 0, :]
    n1 = nf1[0, 0, :]
    a2 = ea2[0, 0, :]
    d2 = ed2[0, 0, :]
    n2 = nf2[0, 0, :]
    xw1_j = (a1[:, None] * w10[None, :] + d1[:, None] * w11[None, :]
             + n1[:, None] * w12[None, :])
    xw1_p = (a2[:, None] * w10[None, :] + d2[:, None] * w11[None, :]
             + n2[:, None] * w12[None, :])

    bf16 = jnp.bfloat16
    b1v = b1[0, 0, :]
    h1_j = jnp.maximum(xw1_j + b1v[None, :], 0.0)
    W2bf = w2[0].astype(bf16)
    h1w2_j = jnp.dot(h1_j.astype(bf16), W2bf,
                     preferred_element_type=f32)

    z = jnp.concatenate([xw1_j, h1w2_j], axis=1)
    agg = jnp.dot(A.astype(bf16), z.astype(bf16), preferred_element_type=f32)

    h1_p = jnp.maximum(dinv2[:, None] * xw1_p + dinv[:, None] * agg[:, :256]
                       + b1v[None, :], 0.0)
    b2v = b2[0, 0, :]
    h2_j = h1w2_j + b2v[None, :]
    h2_p = (dinv2[:, None] * jnp.dot(h1_p.astype(bf16), W2bf,
                                     preferred_element_type=f32)
            + dinv[:, None] * agg[:, 256:] + b2v[None, :])

    c1v = c1[0, 0, :]
    c2v = c2[0, 0, :]
    wo = woutT[0]
    ob = outb[0, 0, :]
    m1bf = m1[0].astype(bf16)
    m2bf = m2[0].astype(bf16)

    def head(h2):
        t1 = jnp.maximum(jnp.dot(h2.astype(bf16), m1bf,
                                 preferred_element_type=f32)
                         + c1v[None, :], 0.0)
        t2 = jnp.maximum(jnp.dot(t1.astype(bf16), m2bf,
                                 preferred_element_type=f32)
                         + c2v[None, :], 0.0)
        s = jnp.sum(t2 * wo, axis=1) + ob
        return 1.0 / (1.0 + jnp.exp(-s))

    out_ref[0, 0, :] = head(h2_j)
    out_ref[0, 1, :] = head(h2_p)


def kernel(actions, node_features, edge_index, emb_table,
           conv1_W, conv1_b, conv2_W, conv2_b,
           mlp1_W, mlp1_b, mlp2_W, mlp2_b, out_W, out_b):
    f32 = jnp.float32
    idx = jnp.concatenate([actions[:, 0], actions[:, 1]]).astype(jnp.int32)
    src = edge_index[:16].reshape(16, EPB)
    dst = edge_index[16:].reshape(16, EPB)

    a_mat, emb_rows = _sc_sparse(src, dst, idx, emb_table)
    emb3 = emb_rows.reshape(64, 1, N)
    nf3 = node_features

    w1 = conv1_W.reshape(1, 3, 256)
    b1 = conv1_b.reshape(1, 1, 256)
    w2 = conv2_W.reshape(1, 256, 128)
    b2 = conv2_b.reshape(1, 1, 128)
    m1 = mlp1_W.reshape(1, 128, 256)
    c1 = mlp1_b.reshape(1, 1, 256)
    m2 = mlp2_W.reshape(1, 256, 256)
    c2 = mlp2_b.reshape(1, 1, 256)
    woutT = out_W.reshape(1, 1, 256)
    outb = jnp.broadcast_to(out_b.reshape(1, 1, 1), (1, 1, N))

    const = lambda j: (0, 0, 0)
    out = pl.pallas_call(
        _tc_body,
        grid=(16,),
        in_specs=[
            pl.BlockSpec((1, 1, N), lambda j: (j, 0, 0)),
            pl.BlockSpec((1, 1, N), lambda j: (j + 32, 0, 0)),
            pl.BlockSpec((1, 1, N), lambda j: (j + 16, 0, 0)),
            pl.BlockSpec((1, 1, N), lambda j: (j + 48, 0, 0)),
            pl.BlockSpec((1, 1, N), lambda j: (j, 0, 0)),
            pl.BlockSpec((1, 1, N), lambda j: (j + 16, 0, 0)),
            pl.BlockSpec((1, N, N), lambda j: (j, 0, 0)),
            pl.BlockSpec((1, 3, 256), const),
            pl.BlockSpec((1, 1, 256), const),
            pl.BlockSpec((1, 256, 128), const),
            pl.BlockSpec((1, 1, 128), const),
            pl.BlockSpec((1, 128, 256), const),
            pl.BlockSpec((1, 1, 256), const),
            pl.BlockSpec((1, 256, 256), const),
            pl.BlockSpec((1, 1, 256), const),
            pl.BlockSpec((1, 1, 256), const),
            pl.BlockSpec((1, 1, N), const),
        ],
        out_specs=pl.BlockSpec((1, 2, N), lambda j: (j, 0, 0)),
        out_shape=jax.ShapeDtypeStruct((16, 2, N), f32),
        compiler_params=pltpu.CompilerParams(
            dimension_semantics=("arbitrary",),
        ),
    )(emb3, emb3, emb3, emb3, nf3, nf3, a_mat,
      w1, b1, w2, b2, m1, c1, m2, c2, woutT, outb)
    return out.transpose(1, 0, 2).reshape(B, N)
